# Initial kernel scaffold; baseline (speedup 1.0000x reference)
import jax
import jax.numpy as jnp
from jax.experimental import pallas as pl


def kernel(logits, labels):
    raise NotImplementedError("write your pallas kernel here")


# speedup vs baseline: 1.0217x; 1.0217x over previous
import jax
import jax.numpy as jnp
from jax.experimental import pallas as pl
from jax.experimental.pallas import tpu as pltpu

S = 64.0
M = 0.4

_ROWS = 16


def _body(lab_ref, x_ref, o_ref):
    x = x_ref[...]
    lab = lab_ref[...]
    cols = jax.lax.broadcasted_iota(jnp.int32, x.shape, 1)
    margin = jnp.where(cols == lab, jnp.float32(-M), jnp.float32(0.0))
    o_ref[...] = (x + margin) * jnp.float32(S)


def kernel(logits, labels):
    B, C = logits.shape
    lab2d = labels.astype(jnp.int32).reshape(B, 1)
    grid = (B // _ROWS,)
    return pl.pallas_call(
        _body,
        grid=grid,
        in_specs=[
            pl.BlockSpec((_ROWS, 1), lambda i: (i, 0)),
            pl.BlockSpec((_ROWS, C), lambda i: (i, 0)),
        ],
        out_specs=pl.BlockSpec((_ROWS, C), lambda i: (i, 0)),
        out_shape=jax.ShapeDtypeStruct((B, C), logits.dtype),
        compiler_params=pltpu.CompilerParams(
            dimension_semantics=("arbitrary",),
        ),
    )(lab2d, logits)


# speedup vs baseline: 1.0253x; 1.0035x over previous
import jax
import jax.numpy as jnp
from jax.experimental import pallas as pl
from jax.experimental.pallas import tpu as pltpu

S = 64.0
M = 0.4

_ROWS = 16


def _body(lab_ref, x_ref, o_ref):
    x = x_ref[...]
    lab = lab_ref[...]
    cols = jax.lax.broadcasted_iota(jnp.int32, x.shape, 1)
    margin = jnp.where(cols == lab, jnp.float32(-M), jnp.float32(0.0))
    o_ref[...] = (x + margin) * jnp.float32(S)


def kernel(logits, labels):
    B, C = logits.shape
    lab2d = labels.astype(jnp.int32).reshape(B, 1)
    grid = (B // _ROWS,)
    return pl.pallas_call(
        _body,
        grid=grid,
        in_specs=[
            pl.BlockSpec((_ROWS, 1), lambda i: (i, 0)),
            pl.BlockSpec((_ROWS, C), lambda i: (i, 0)),
        ],
        out_specs=pl.BlockSpec((_ROWS, C), lambda i: (i, 0)),
        out_shape=jax.ShapeDtypeStruct((B, C), logits.dtype),
        compiler_params=pltpu.CompilerParams(
            dimension_semantics=("parallel",),
        ),
    )(lab2d, logits)


# speedup vs baseline: 1.0264x; 1.0010x over previous
import jax
import jax.numpy as jnp
from jax.experimental import pallas as pl
from jax.experimental.pallas import tpu as pltpu

S = 64.0
M = 0.4

_ROWS = 8
_DEPTH = 6


def _body(lab_ref, x_hbm, o_hbm, xbuf, obuf, insem, outsem):
    n_chunks = x_hbm.shape[0] // _ROWS

    def in_cp(i, s):
        return pltpu.make_async_copy(
            x_hbm.at[pl.ds(i * _ROWS, _ROWS), :], xbuf.at[s], insem.at[s])

    def out_cp(i, s):
        return pltpu.make_async_copy(
            obuf.at[s], o_hbm.at[pl.ds(i * _ROWS, _ROWS), :], outsem.at[s])

    for s in range(_DEPTH):
        in_cp(s, s).start()

    def step(i, carry):
        s = jax.lax.rem(i, _DEPTH)
        in_cp(i, s).wait()

        @pl.when(i >= _DEPTH)
        def _():
            out_cp(i - _DEPTH, s).wait()

        x = xbuf[s]
        lab = lab_ref[pl.ds(i * _ROWS, _ROWS), :]
        cols = jax.lax.broadcasted_iota(jnp.int32, x.shape, 1)
        margin = jnp.where(cols == lab, jnp.float32(-M), jnp.float32(0.0))
        obuf[s] = (x + margin) * jnp.float32(S)

        out_cp(i, s).start()

        @pl.when(i + _DEPTH < n_chunks)
        def _():
            in_cp(i + _DEPTH, s).start()

        return carry

    jax.lax.fori_loop(0, n_chunks, step, 0)

    for s in range(_DEPTH):
        out_cp(n_chunks - _DEPTH + s, s).wait()


def kernel(logits, labels):
    B, C = logits.shape
    lab2d = labels.astype(jnp.int32).reshape(B, 1)
    return pl.pallas_call(
        _body,
        in_specs=[
            pl.BlockSpec(memory_space=pltpu.VMEM),
            pl.BlockSpec(memory_space=pl.ANY),
        ],
        out_specs=pl.BlockSpec(memory_space=pl.ANY),
        out_shape=jax.ShapeDtypeStruct((B, C), logits.dtype),
        scratch_shapes=[
            pltpu.VMEM((_DEPTH, _ROWS, C), logits.dtype),
            pltpu.VMEM((_DEPTH, _ROWS, C), logits.dtype),
            pltpu.SemaphoreType.DMA((_DEPTH,)),
            pltpu.SemaphoreType.DMA((_DEPTH,)),
        ],
    )(lab2d, logits)
